# Initial kernel scaffold; baseline (speedup 1.0000x reference)
import jax
import jax.numpy as jnp
from jax.experimental import pallas as pl


def kernel(x, _as, _au, S, Phi, A, II, initial_state):
    raise NotImplementedError("write your pallas kernel here")


# speedup vs baseline: 252.7237x; 252.7237x over previous
import functools

import jax
import jax.numpy as jnp
from jax import lax
from jax.experimental import pallas as pl
from jax.experimental.pallas import tpu as pltpu
from jax.experimental.pallas import tpu_sc as plsc

NSTATE = 64
TLEN = 2048
LANES = 16


def _sc_body(x_hbm, consts_hbm, svec_hbm, avec_hbm, iivec_hbm, s0_hbm,
             out_hbm, x_v, base_v, sv, av, iv, s0v, cv, outv):
    num_cores = plsc.get_sparse_core_info().num_cores
    wid = lax.axis_index("s") * num_cores + lax.axis_index("c")

    @pl.when(wid == 0)
    def _():
        pltpu.sync_copy(x_hbm, x_v)
        pltpu.sync_copy(consts_hbm, cv)
        pltpu.sync_copy(svec_hbm, sv)
        pltpu.sync_copy(avec_hbm, av)
        pltpu.sync_copy(iivec_hbm, iv)
        pltpu.sync_copy(s0_hbm, s0v)

        asv = cv[pl.ds(0, LANES)]
        auv = cv[pl.ds(LANES, LANES)]
        phiv = cv[pl.ds(2 * LANES, LANES)]

        lane = lax.iota(jnp.int32, LANES)

        def lanesum(v):
            for sh in (8, 4, 2, 1):
                v = v + v.at[lane ^ sh].get(mode="promise_in_bounds")
            return v

        pp = jnp.zeros((LANES,), jnp.float32)
        qq = jnp.zeros((LANES,), jnp.float32)
        rr = jnp.zeros((LANES,), jnp.float32)
        dd = jnp.zeros((LANES,), jnp.float32)
        ww = jnp.zeros((LANES,), jnp.float32)
        for j in range(NSTATE // LANES):
            s_c = sv[pl.ds(j * LANES, LANES)]
            a_c = av[pl.ds(j * LANES, LANES)]
            i_c = iv[pl.ds(j * LANES, LANES)]
            s0_c = s0v[pl.ds(j * LANES, LANES)]
            pp = pp + s_c * s_c
            qq = qq + s_c * a_c
            rr = rr + a_c * a_c
            ds = s_c - s0_c
            dd = dd + ds * ds
            ww = ww + i_c * a_c

        pv = lanesum(pp)
        q2v = lanesum(qq)
        rv = lanesum(rr)
        d0v = lanesum(dd)
        wv = lanesum(ww)

        k0v = -asv * pv
        k1v = asv * (q2v + q2v)
        k2v = -asv * rv

        def base_step(i, c):
            xc = x_v[pl.ds(i * LANES, LANES)]
            dx = phiv - xc
            base_v[pl.ds(i * LANES, LANES)] = k0v - auv * dx * dx
            return c

        lax.fori_loop(0, TLEN // LANES, base_step, 0)

        b0 = base_v[pl.ds(0, LANES)]
        y0 = b0 - k0v - asv * d0v

        def rec_step(t, y):
            e = jnp.exp(y)
            bt = base_v[pl.ds(t, LANES)]
            return bt + e * (k1v + k2v * e)

        y = lax.fori_loop(1, TLEN, rec_step, y0)

        outv[...] = jnp.exp(y) * wv
        pltpu.sync_copy(outv, out_hbm)


@jax.jit
def _run(x_flat, consts, svec, avec, iivec, s0vec):
    mesh = plsc.VectorSubcoreMesh(core_axis_name="c", subcore_axis_name="s")
    f = functools.partial(
        pl.kernel,
        mesh=mesh,
        out_type=jax.ShapeDtypeStruct((LANES,), jnp.float32),
        scratch_types=[
            pltpu.VMEM((TLEN,), jnp.float32),
            pltpu.VMEM((TLEN + LANES,), jnp.float32),
            pltpu.VMEM((NSTATE,), jnp.float32),
            pltpu.VMEM((NSTATE,), jnp.float32),
            pltpu.VMEM((NSTATE,), jnp.float32),
            pltpu.VMEM((NSTATE,), jnp.float32),
            pltpu.VMEM((3 * LANES,), jnp.float32),
            pltpu.VMEM((LANES,), jnp.float32),
        ],
    )(_sc_body)
    return f(x_flat, consts, svec, avec, iivec, s0vec)


def kernel(x, _as, _au, S, Phi, A, II, initial_state):
    x_flat = x.reshape(TLEN)
    consts = jnp.concatenate([
        jnp.broadcast_to(_as.reshape(1), (LANES,)),
        jnp.broadcast_to(_au.reshape(1), (LANES,)),
        jnp.broadcast_to(Phi.reshape(1), (LANES,)),
    ])
    out16 = _run(x_flat, consts, S.reshape(NSTATE), A.reshape(NSTATE),
                 II.reshape(NSTATE), initial_state.reshape(NSTATE))
    return out16[:1].reshape(1, 1)


# speedup vs baseline: 267.6321x; 1.0590x over previous
import functools

import jax
import jax.numpy as jnp
from jax import lax
from jax.experimental import pallas as pl
from jax.experimental.pallas import tpu as pltpu
from jax.experimental.pallas import tpu_sc as plsc

NSTATE = 64
TLEN = 2048
LANES = 16
UNROLL = 23


def _sc_body(x_hbm, consts_hbm, svec_hbm, avec_hbm, iivec_hbm, s0_hbm,
             out_hbm, x_v, base_v, sv, av, iv, s0v, cv, outv):
    num_cores = plsc.get_sparse_core_info().num_cores
    wid = lax.axis_index("s") * num_cores + lax.axis_index("c")

    @pl.when(wid == 0)
    def _():
        pltpu.sync_copy(x_hbm, x_v)
        pltpu.sync_copy(consts_hbm, cv)
        pltpu.sync_copy(svec_hbm, sv)
        pltpu.sync_copy(avec_hbm, av)
        pltpu.sync_copy(iivec_hbm, iv)
        pltpu.sync_copy(s0_hbm, s0v)

        asv = cv[pl.ds(0, LANES)]
        auv = cv[pl.ds(LANES, LANES)]
        phiv = cv[pl.ds(2 * LANES, LANES)]

        lane = lax.iota(jnp.int32, LANES)

        def lanesum(v):
            for sh in (8, 4, 2, 1):
                v = v + v.at[lane ^ sh].get(mode="promise_in_bounds")
            return v

        pp = jnp.zeros((LANES,), jnp.float32)
        qq = jnp.zeros((LANES,), jnp.float32)
        rr = jnp.zeros((LANES,), jnp.float32)
        dd = jnp.zeros((LANES,), jnp.float32)
        ww = jnp.zeros((LANES,), jnp.float32)
        for j in range(NSTATE // LANES):
            s_c = sv[pl.ds(j * LANES, LANES)]
            a_c = av[pl.ds(j * LANES, LANES)]
            i_c = iv[pl.ds(j * LANES, LANES)]
            s0_c = s0v[pl.ds(j * LANES, LANES)]
            pp = pp + s_c * s_c
            qq = qq + s_c * a_c
            rr = rr + a_c * a_c
            ds = s_c - s0_c
            dd = dd + ds * ds
            ww = ww + i_c * a_c

        pv = lanesum(pp)
        q2v = lanesum(qq)
        rv = lanesum(rr)
        d0v = lanesum(dd)
        wv = lanesum(ww)

        k0v = -asv * pv
        k1v = asv * (q2v + q2v)
        k2v = -asv * rv

        def base_step(i, c):
            xc = x_v[pl.ds(i * LANES, LANES)]
            dx = phiv - xc
            base_v[pl.ds(i * LANES, LANES)] = k0v - auv * dx * dx
            return c

        lax.fori_loop(0, TLEN // LANES, base_step, 0)

        b0 = base_v[pl.ds(0, LANES)]
        y0 = b0 - k0v - asv * d0v

        def rec_block(i, y):
            t0 = 1 + i * UNROLL
            for j in range(UNROLL):
                e = jnp.exp(y)
                v = e * e
                bt = base_v[pl.ds(t0 + j, LANES)]
                y = (bt + k1v * e) + k2v * v
            return y

        y = lax.fori_loop(0, (TLEN - 1) // UNROLL, rec_block, y0)

        outv[...] = jnp.exp(y) * wv
        pltpu.sync_copy(outv, out_hbm)


@jax.jit
def _run(x_flat, consts, svec, avec, iivec, s0vec):
    mesh = plsc.VectorSubcoreMesh(core_axis_name="c", subcore_axis_name="s")
    f = functools.partial(
        pl.kernel,
        mesh=mesh,
        out_type=jax.ShapeDtypeStruct((LANES,), jnp.float32),
        scratch_types=[
            pltpu.VMEM((TLEN,), jnp.float32),
            pltpu.VMEM((TLEN + LANES,), jnp.float32),
            pltpu.VMEM((NSTATE,), jnp.float32),
            pltpu.VMEM((NSTATE,), jnp.float32),
            pltpu.VMEM((NSTATE,), jnp.float32),
            pltpu.VMEM((NSTATE,), jnp.float32),
            pltpu.VMEM((3 * LANES,), jnp.float32),
            pltpu.VMEM((LANES,), jnp.float32),
        ],
    )(_sc_body)
    return f(x_flat, consts, svec, avec, iivec, s0vec)


def kernel(x, _as, _au, S, Phi, A, II, initial_state):
    x_flat = x.reshape(TLEN)
    consts = jnp.concatenate([
        jnp.broadcast_to(_as.reshape(1), (LANES,)),
        jnp.broadcast_to(_au.reshape(1), (LANES,)),
        jnp.broadcast_to(Phi.reshape(1), (LANES,)),
    ])
    out16 = _run(x_flat, consts, S.reshape(NSTATE), A.reshape(NSTATE),
                 II.reshape(NSTATE), initial_state.reshape(NSTATE))
    return out16[:1].reshape(1, 1)


# speedup vs baseline: 282.5697x; 1.0558x over previous
import functools

import jax
import jax.numpy as jnp
from jax import lax
from jax.experimental import pallas as pl
from jax.experimental.pallas import tpu as pltpu
from jax.experimental.pallas import tpu_sc as plsc

NSTATE = 64
TLEN = 2048
LANES = 16
UNROLL = 23


def _sc_body(x_hbm, pk_hbm, out_hbm, x_v, pk_v, outv, sem):
    num_cores = plsc.get_sparse_core_info().num_cores
    wid = lax.axis_index("s") * num_cores + lax.axis_index("c")

    @pl.when(wid == 0)
    def _():
        cx = pltpu.async_copy(x_hbm, x_v.at[pl.ds(0, TLEN)], sem)
        cp = pltpu.async_copy(pk_hbm, pk_v, sem)
        cx.wait()
        cp.wait()

        asv = pk_v[pl.ds(4 * NSTATE, LANES)]
        auv = pk_v[pl.ds(4 * NSTATE + LANES, LANES)]
        phiv = pk_v[pl.ds(4 * NSTATE + 2 * LANES, LANES)]

        lane = lax.iota(jnp.int32, LANES)

        def lanesum(v):
            for sh in (8, 4, 2, 1):
                v = v + v.at[lane ^ sh].get(mode="promise_in_bounds")
            return v

        pp = jnp.zeros((LANES,), jnp.float32)
        qq = jnp.zeros((LANES,), jnp.float32)
        rr = jnp.zeros((LANES,), jnp.float32)
        dd = jnp.zeros((LANES,), jnp.float32)
        ww = jnp.zeros((LANES,), jnp.float32)
        for j in range(NSTATE // LANES):
            s_c = pk_v[pl.ds(j * LANES, LANES)]
            a_c = pk_v[pl.ds(NSTATE + j * LANES, LANES)]
            i_c = pk_v[pl.ds(2 * NSTATE + j * LANES, LANES)]
            s0_c = pk_v[pl.ds(3 * NSTATE + j * LANES, LANES)]
            pp = pp + s_c * s_c
            qq = qq + s_c * a_c
            rr = rr + a_c * a_c
            ds = s_c - s0_c
            dd = dd + ds * ds
            ww = ww + i_c * a_c

        pv = lanesum(pp)
        q2v = lanesum(qq)
        rv = lanesum(rr)
        d0v = lanesum(dd)
        wv = lanesum(ww)

        k0v = -asv * pv
        k1v = asv * (q2v + q2v)
        k2v = -asv * rv

        dx0 = phiv - x_v[pl.ds(0, LANES)]
        y0 = -auv * dx0 * dx0 - asv * d0v

        def rec_block(i, y):
            t0 = 1 + i * UNROLL
            for j in range(UNROLL):
                e = jnp.exp(y)
                v = e * e
                xc = x_v[pl.ds(t0 + j, LANES)]
                dxt = phiv - xc
                bt = k0v - auv * dxt * dxt
                y = (bt + k1v * e) + k2v * v
            return y

        y = lax.fori_loop(0, (TLEN - 1) // UNROLL, rec_block, y0)

        outv[...] = jnp.exp(y) * wv
        pltpu.sync_copy(outv, out_hbm)


@jax.jit
def _run(x_flat, packed):
    mesh = plsc.VectorSubcoreMesh(core_axis_name="c", subcore_axis_name="s")
    f = functools.partial(
        pl.kernel,
        mesh=mesh,
        out_type=jax.ShapeDtypeStruct((LANES,), jnp.float32),
        scratch_types=[
            pltpu.VMEM((TLEN + LANES,), jnp.float32),
            pltpu.VMEM((4 * NSTATE + 3 * LANES,), jnp.float32),
            pltpu.VMEM((LANES,), jnp.float32),
            pltpu.SemaphoreType.DMA,
        ],
    )(_sc_body)
    return f(x_flat, packed)


def kernel(x, _as, _au, S, Phi, A, II, initial_state):
    x_flat = x.reshape(TLEN)
    packed = jnp.concatenate([
        S.reshape(NSTATE),
        A.reshape(NSTATE),
        II.reshape(NSTATE),
        initial_state.reshape(NSTATE),
        jnp.broadcast_to(_as.reshape(1), (LANES,)),
        jnp.broadcast_to(_au.reshape(1), (LANES,)),
        jnp.broadcast_to(Phi.reshape(1), (LANES,)),
    ])
    out16 = _run(x_flat, packed)
    return out16[:1].reshape(1, 1)
